# baseline (speedup 1.0000x reference)
import jax
import jax.numpy as jnp
from jax.experimental import pallas as pl
from jax.experimental.pallas import tpu as pltpu

_BM_TARGET = 200
_ADJ_BUFFERS = 6


def _pick_bm(n: int, target: int) -> int:
    best = 8
    for d in range(8, target + 1, 8):
        if n % d == 0:
            best = d
    return best


def _make_outer_kernel(bm: int, n: int, n_class: int):
    nb = n // bm

    def _outer(adj_hbm, x_ref, w0_ref, b0_ref, w1_ref, b1_ref,
               out_hbm, s0_scratch, s1_scratch, step_ref):
        step_ref[0] = 0
        x = jnp.maximum(x_ref[...], 0.0)
        s0_scratch[...] = (
            jnp.dot(x, w0_ref[...], preferred_element_type=jnp.float32)
            + b0_ref[...]
        )

        def _body(adj_blk, out_blk):
            step = step_ref[0]
            ph = step // nb
            i = step % nb
            step_ref[0] = step + 1

            @pl.when(ph == 0)
            def _():
                acc = jnp.dot(adj_blk[...], s0_scratch[...],
                              preferred_element_type=jnp.float32)
                x1 = jnp.maximum(acc, 0.0)
                s1_blk = (
                    jnp.dot(x1, w1_ref[...],
                            preferred_element_type=jnp.float32)
                    + b1_ref[...]
                )
                s1_scratch[pl.ds(i * bm, bm), :] = s1_blk

            @pl.when(ph == 1)
            def _():
                acc = jnp.dot(adj_blk[...], s1_scratch[...],
                              preferred_element_type=jnp.float32)
                x2 = jnp.maximum(acc, 0.0)
                m = jnp.max(x2, axis=1, keepdims=True)
                z = x2 - m
                lse = jnp.log(jnp.sum(jnp.exp(z), axis=1, keepdims=True))
                out_blk[...] = z - lse

        pipeline = pltpu.emit_pipeline(
            _body,
            grid=(2, nb),
            in_specs=[
                pl.BlockSpec((bm, n),
                             lambda ph, i: (i + ph * (nb - 1 - 2 * i), 0),
                             pipeline_mode=pl.Buffered(
                                 buffer_count=_ADJ_BUFFERS,
                                 use_lookahead=True)),
            ],
            out_specs=[
                pl.BlockSpec((bm, n_class),
                             lambda ph, i: (nb - 1 - ph * i, 0)),
            ],
        )
        pipeline(adj_hbm, out_hbm)

    return _outer


@jax.jit
def kernel(input, adj, W0, b0, W1, b1):
    n, in_size = input.shape
    hidd = W0.shape[1]
    n_class = W1.shape[1]
    bm = _pick_bm(n, _BM_TARGET)

    b0_2d = b0.reshape(1, hidd)
    b1_2d = b1.reshape(1, n_class)

    vmem_full = pl.BlockSpec(memory_space=pltpu.VMEM)

    out = pl.pallas_call(
        _make_outer_kernel(bm, n, n_class),
        in_specs=[
            pl.BlockSpec(memory_space=pl.ANY),
            vmem_full, vmem_full, vmem_full, vmem_full, vmem_full,
        ],
        out_specs=pl.BlockSpec(memory_space=pl.ANY),
        out_shape=jax.ShapeDtypeStruct((n, n_class), jnp.float32),
        compiler_params=pltpu.CompilerParams(
            vmem_limit_bytes=100 * 1024 * 1024),
        scratch_shapes=[
            pltpu.VMEM((n, hidd), jnp.float32),
            pltpu.VMEM((n, n_class), jnp.float32),
            pltpu.SMEM((1,), jnp.int32),
        ],
    )(adj, input, W0, b0_2d, W1, b1_2d)

    return out


# speedup vs baseline: 1.0173x; 1.0173x over previous
import jax
import jax.numpy as jnp
from jax.experimental import pallas as pl
from jax.experimental.pallas import tpu as pltpu

_BM_TARGET = 200
_ADJ_BUFFERS = 4


def _pick_bm(n: int, target: int) -> int:
    best = 8
    for d in range(8, target + 1, 8):
        if n % d == 0:
            best = d
    return best


def _make_outer_kernel(bm: int, n: int, n_class: int):
    nb = n // bm

    def _outer(adj_hbm, x_ref, w0_ref, b0_ref, w1_ref, b1_ref,
               out_hbm, s0_scratch, s1_scratch, step_ref):
        step_ref[0] = 0
        x = jnp.maximum(x_ref[...], 0.0)
        s0_scratch[...] = (
            jnp.dot(x, w0_ref[...], preferred_element_type=jnp.float32)
            + b0_ref[...]
        )

        def _body(adj_blk, out_blk):
            step = step_ref[0]
            ph = step // nb
            i = step % nb
            step_ref[0] = step + 1

            @pl.when(ph == 0)
            def _():
                acc = jnp.dot(adj_blk[...], s0_scratch[...],
                              preferred_element_type=jnp.float32)
                x1 = jnp.maximum(acc, 0.0)
                s1_blk = (
                    jnp.dot(x1, w1_ref[...],
                            preferred_element_type=jnp.float32)
                    + b1_ref[...]
                )
                s1_scratch[pl.ds(i * bm, bm), :] = s1_blk

            @pl.when(ph == 1)
            def _():
                acc = jnp.dot(adj_blk[...], s1_scratch[...],
                              preferred_element_type=jnp.float32)
                x2 = jnp.maximum(acc, 0.0)
                m = jnp.max(x2, axis=1, keepdims=True)
                z = x2 - m
                lse = jnp.log(jnp.sum(jnp.exp(z), axis=1, keepdims=True))
                out_blk[...] = z - lse

        pipeline = pltpu.emit_pipeline(
            _body,
            grid=(2, nb),
            in_specs=[
                pl.BlockSpec((bm, n),
                             lambda ph, i: (i + ph * (nb - 1 - 2 * i), 0),
                             pipeline_mode=pl.Buffered(
                                 buffer_count=_ADJ_BUFFERS,
                                 use_lookahead=True)),
            ],
            out_specs=[
                pl.BlockSpec((bm, n_class),
                             lambda ph, i: (nb - 1 - ph * i, 0)),
            ],
        )
        pipeline(adj_hbm, out_hbm)

    return _outer


@jax.jit
def kernel(input, adj, W0, b0, W1, b1):
    n, in_size = input.shape
    hidd = W0.shape[1]
    n_class = W1.shape[1]
    bm = _pick_bm(n, _BM_TARGET)

    b0_2d = b0.reshape(1, hidd)
    b1_2d = b1.reshape(1, n_class)

    vmem_full = pl.BlockSpec(memory_space=pltpu.VMEM)

    out = pl.pallas_call(
        _make_outer_kernel(bm, n, n_class),
        in_specs=[
            pl.BlockSpec(memory_space=pl.ANY),
            vmem_full, vmem_full, vmem_full, vmem_full, vmem_full,
        ],
        out_specs=pl.BlockSpec(memory_space=pl.ANY),
        out_shape=jax.ShapeDtypeStruct((n, n_class), jnp.float32),
        scratch_shapes=[
            pltpu.VMEM((n, hidd), jnp.float32),
            pltpu.VMEM((n, n_class), jnp.float32),
            pltpu.SMEM((1,), jnp.int32),
        ],
    )(adj, input, W0, b0_2d, W1, b1_2d)

    return out


# speedup vs baseline: 1.0199x; 1.0026x over previous
import jax
import jax.numpy as jnp
from jax.experimental import pallas as pl
from jax.experimental.pallas import tpu as pltpu

_BM_TARGET = 200
_ADJ_BUFFERS = 4


def _pick_bm(n: int, target: int) -> int:
    best = 8
    for d in range(8, target + 1, 8):
        if n % d == 0:
            best = d
    return best


def _make_outer_kernel(bm: int, n: int, n_class: int):
    nb = n // bm

    def _outer(adj_hbm, x_ref, w0_ref, b0_ref, w1_ref, b1_ref,
               out_hbm, s0_scratch, s1_scratch, step_ref):
        step_ref[0] = 0

        def _body(adj_blk, out_blk):
            step = step_ref[0]
            ph = step // nb
            i = step % nb
            step_ref[0] = step + 1

            @pl.when(step == 0)
            def _():
                x = jnp.maximum(x_ref[...], 0.0)
                s0_scratch[...] = (
                    jnp.dot(x, w0_ref[...],
                            preferred_element_type=jnp.float32)
                    + b0_ref[...]
                )

            @pl.when(ph == 0)
            def _():
                acc = jnp.dot(adj_blk[...], s0_scratch[...],
                              preferred_element_type=jnp.float32)
                x1 = jnp.maximum(acc, 0.0)
                s1_blk = (
                    jnp.dot(x1, w1_ref[...],
                            preferred_element_type=jnp.float32)
                    + b1_ref[...]
                )
                s1_scratch[pl.ds(i * bm, bm), :] = s1_blk

            @pl.when(ph == 1)
            def _():
                acc = jnp.dot(adj_blk[...], s1_scratch[...],
                              preferred_element_type=jnp.float32)
                x2 = jnp.maximum(acc, 0.0)
                m = jnp.max(x2, axis=1, keepdims=True)
                z = x2 - m
                lse = jnp.log(jnp.sum(jnp.exp(z), axis=1, keepdims=True))
                out_blk[...] = z - lse

        pipeline = pltpu.emit_pipeline(
            _body,
            grid=(2, nb),
            in_specs=[
                pl.BlockSpec((bm, n),
                             lambda ph, i: (i + ph * (nb - 1 - 2 * i), 0),
                             pipeline_mode=pl.Buffered(
                                 buffer_count=_ADJ_BUFFERS,
                                 use_lookahead=True)),
            ],
            out_specs=[
                pl.BlockSpec((bm, n_class),
                             lambda ph, i: (nb - 1 - ph * i, 0)),
            ],
        )
        pipeline(adj_hbm, out_hbm)

    return _outer


@jax.jit
def kernel(input, adj, W0, b0, W1, b1):
    n, in_size = input.shape
    hidd = W0.shape[1]
    n_class = W1.shape[1]
    bm = _pick_bm(n, _BM_TARGET)

    b0_2d = b0.reshape(1, hidd)
    b1_2d = b1.reshape(1, n_class)

    vmem_full = pl.BlockSpec(memory_space=pltpu.VMEM)

    out = pl.pallas_call(
        _make_outer_kernel(bm, n, n_class),
        in_specs=[
            pl.BlockSpec(memory_space=pl.ANY),
            vmem_full, vmem_full, vmem_full, vmem_full, vmem_full,
        ],
        out_specs=pl.BlockSpec(memory_space=pl.ANY),
        out_shape=jax.ShapeDtypeStruct((n, n_class), jnp.float32),
        scratch_shapes=[
            pltpu.VMEM((n, hidd), jnp.float32),
            pltpu.VMEM((n, n_class), jnp.float32),
            pltpu.SMEM((1,), jnp.int32),
        ],
    )(adj, input, W0, b0_2d, W1, b1_2d)

    return out
